# baseline (speedup 1.0000x reference)
import functools

import jax
import jax.numpy as jnp
from jax import lax
from jax.experimental import pallas as pl
from jax.experimental.pallas import tpu as pltpu
from jax.experimental.pallas import tpu_sc as plsc

NUM_CODES = 8192
LATENT_DIM = 256
BETA = 0.25

TM = 256

_WINDOW = 2736


def _assign_body(z_ref, e_ref, zn_ref, en_ref, idx_ref, loss_ref):
    t = pl.program_id(0)

    z = z_ref[...]
    e = e_ref[...]
    scores = lax.dot_general(z, e, (((1,), (1,)), ((), ())),
                             preferred_element_type=jnp.float32)
    znorm = zn_ref[...]
    enorm = en_ref[...]
    d = (znorm + enorm) - 2.0 * scores

    ids = lax.broadcasted_iota(jnp.int32, d.shape, 1)
    inf = jnp.float32(jnp.inf)
    imax = jnp.int32(2**31 - 1)

    m_cmp = None
    m_true = None
    m_idx = None
    for lo in range(0, NUM_CODES, _WINDOW):
        hi = min(lo + _WINDOW, NUM_CODES)
        maskw = (ids >= lo) & (ids < hi)
        dw = jnp.where(maskw, d, inf)
        mw = jnp.min(dw, axis=1, keepdims=True)
        iw = jnp.min(jnp.where(dw == mw, ids, imax),
                     axis=1, keepdims=True)
        if m_cmp is None:
            m_cmp, m_true, m_idx = mw, mw, iw
        else:
            keep = (m_cmp < mw) | ((m_cmp == mw) & (m_idx < iw))
            m_cmp = jnp.where(keep, m_cmp, mw)
            m_true = jnp.where(keep, m_true, mw)
            m_idx = jnp.where(keep, m_idx, iw)
        m_cmp = m_cmp.astype(jnp.bfloat16).astype(jnp.float32)

    idx_ref[...] = m_idx
    s = jnp.sum(m_true, axis=(0, 1), keepdims=True)

    @pl.when(t == 0)
    def _():
        loss_ref[...] = s

    @pl.when(t > 0)
    def _():
        loss_ref[...] += s


def _assign(zt, codebook, znorm, enorm):
    n = zt.shape[0]
    grid = (n // TM,)
    return pl.pallas_call(
        _assign_body,
        grid=grid,
        in_specs=[
            pl.BlockSpec((TM, LATENT_DIM), lambda t: (t, 0)),
            pl.BlockSpec((NUM_CODES, LATENT_DIM), lambda t: (0, 0)),
            pl.BlockSpec((TM, 1), lambda t: (t, 0)),
            pl.BlockSpec((1, NUM_CODES), lambda t: (0, 0)),
        ],
        out_specs=[
            pl.BlockSpec((TM, 1), lambda t: (t, 0)),
            pl.BlockSpec((1, 1), lambda t: (0, 0)),
        ],
        out_shape=[
            jax.ShapeDtypeStruct((n, 1), jnp.int32),
            jax.ShapeDtypeStruct((1, 1), jnp.float32),
        ],
    )(zt, codebook, znorm, enorm)


_SC_CHUNK = 128


def _make_gather(n_tokens):
    info = plsc.get_sparse_core_info()
    ncores, nsub = info.num_cores, info.num_subcores
    nw = ncores * nsub
    b_per_w = n_tokens // nw
    n_chunks = b_per_w // _SC_CHUNK
    mesh = plsc.VectorSubcoreMesh(core_axis_name="c", subcore_axis_name="s")

    @functools.partial(
        pl.kernel,
        mesh=mesh,
        out_type=jax.ShapeDtypeStruct((n_tokens, LATENT_DIM), jnp.float32),
        scratch_types=[
            pltpu.VMEM((_SC_CHUNK,), jnp.int32),
            pltpu.VMEM((_SC_CHUNK, LATENT_DIM), jnp.float32),
            pltpu.SemaphoreType.DMA,
        ],
    )
    def gather_k(table_hbm, idx_hbm, out_hbm, idx_v, rows_v, sem):
        wid = lax.axis_index("s") * ncores + lax.axis_index("c")
        base = wid * b_per_w
        for j in range(n_chunks):
            off = base + j * _SC_CHUNK
            pltpu.sync_copy(idx_hbm.at[pl.ds(off, _SC_CHUNK)], idx_v)
            pltpu.async_copy(table_hbm.at[idx_v], rows_v, sem).wait()
            pltpu.sync_copy(rows_v, out_hbm.at[pl.ds(off, _SC_CHUNK)])

    return gather_k


def kernel(z, codebook):
    b, ch, h, w = z.shape
    n = b * h * w
    zt = jnp.transpose(z, (0, 2, 3, 1)).reshape(n, ch)
    znorm = jnp.sum(zt ** 2, axis=1, keepdims=True)
    enorm = jnp.sum(codebook ** 2, axis=1).reshape(1, NUM_CODES)
    idx2, dmin_sum = _assign(zt, codebook, znorm, enorm)
    cb_u = jax.lax.bitcast_convert_type(codebook, jnp.uint32)
    cb_u = (cb_u + jnp.uint32(0x7FFF) + ((cb_u >> 16) & jnp.uint32(1))) \
        & jnp.uint32(0xFFFF0000)
    cb_q = jax.lax.bitcast_convert_type(cb_u, jnp.float32)
    zq_flat = _make_gather(n)(cb_q, idx2.reshape(n))
    loss = dmin_sum[0, 0] * (BETA / (n * ch))
    zt4 = jnp.transpose(z, (0, 2, 3, 1))
    zq4 = zq_flat.reshape(b, h, w, ch)
    zq4 = zt4 + (zq4 - zt4)
    z_q = jnp.transpose(zq4, (0, 3, 1, 2))
    return (z_q, idx2, loss)
